# baseline (speedup 1.0000x reference)
import jax
import jax.numpy as jnp
from jax.experimental import pallas as pl
from jax.experimental.pallas import tpu as pltpu


def _mlp_body(h_ref, wl1_ref, bl1_ref, wl2_ref, bl2_ref, o_ref):
    h = h_ref[...]
    t = jnp.maximum(jnp.dot(h, wl1_ref[...], preferred_element_type=jnp.float32)
                    + bl1_ref[...], 0.0)
    o = jnp.maximum(jnp.dot(t, wl2_ref[...], preferred_element_type=jnp.float32)
                    + bl2_ref[...], 0.0)
    o_ref[...] = o


def _mlp(h, Wl1, bl1, Wl2, bl2):
    n, d = h.shape
    bn = 1000
    grid = (n // bn,)
    return pl.pallas_call(
        _mlp_body,
        grid=grid,
        in_specs=[
            pl.BlockSpec((bn, d), lambda i: (i, 0)),
            pl.BlockSpec(Wl1.shape, lambda i: (0, 0)),
            pl.BlockSpec(bl1.shape, lambda i: (0,)),
            pl.BlockSpec(Wl2.shape, lambda i: (0, 0)),
            pl.BlockSpec(bl2.shape, lambda i: (0,)),
        ],
        out_specs=pl.BlockSpec((bn, Wl2.shape[1]), lambda i: (i, 0)),
        out_shape=jax.ShapeDtypeStruct((n, Wl2.shape[1]), jnp.float32),
    )(h, Wl1, bl1, Wl2, bl2)


def _gcn_conv(x, edge_index, edge_weight, W, b):
    n = x.shape[0]
    x = x @ W
    loop = jnp.arange(n, dtype=edge_index.dtype)
    row = jnp.concatenate([edge_index[0], loop])
    col = jnp.concatenate([edge_index[1], loop])
    ew = jnp.concatenate([edge_weight, jnp.ones((n,), dtype=x.dtype)])
    deg = jax.ops.segment_sum(ew, col, num_segments=n)
    safe_deg = jnp.where(deg > 0, deg, 1.0)
    dinv = jnp.where(deg > 0, 1.0 / jnp.sqrt(safe_deg), 0.0)
    norm = dinv[row] * ew * dinv[col]
    out = jax.ops.segment_sum(norm[:, None] * x[row], col, num_segments=n)
    return out + b


def kernel(x, edge_index, edge_weight, W1, b1, W2, b2, Wl1, bl1, Wl2, bl2):
    h = _gcn_conv(x, edge_index, edge_weight, W1, b1)
    h = jax.nn.relu(h)
    h = _gcn_conv(h, edge_index, edge_weight, W2, b2)
    h = jax.nn.relu(h)
    return _mlp(h, Wl1, bl1, Wl2, bl2)


# speedup vs baseline: 10.8595x; 10.8595x over previous
import functools

import jax
import jax.numpy as jnp
from jax import lax
from jax.experimental import pallas as pl
from jax.experimental.pallas import tpu as pltpu
from jax.experimental.pallas import tpu_sc as plsc

N = 10000
E = 320000
NC = 2
NS = 16
NP = 10240

DEG_CH, DEG_EPB = 125, 80
EPB = 80
RP_S = NP // NS
RP_BLK = 64
MEGAS = 5

_mesh = plsc.VectorSubcoreMesh(core_axis_name="c", subcore_axis_name="s")


@functools.partial(
    pl.kernel,
    out_type=jax.ShapeDtypeStruct((NC * NP,), jnp.float32),
    mesh=_mesh,
    compiler_params=pltpu.CompilerParams(needs_layout_passes=False),
    scratch_types=[
        pltpu.VMEM((DEG_CH, DEG_EPB), jnp.int32),
        pltpu.VMEM((DEG_CH * DEG_EPB,), jnp.float32),
        pltpu.VMEM((NP // NS,), jnp.float32),
        pltpu.VMEM_SHARED((NP,), jnp.float32),
    ],
)
def _deg(col_hbm, ew_hbm, deg_hbm, colb, ewb, zb, deg_sh):
    c = lax.axis_index("c")
    s = lax.axis_index("s")
    npw = NP // NS

    def zero(i, carry):
        zb[pl.ds(i * 16, 16)] = jnp.zeros((16,), jnp.float32)
        return carry

    lax.fori_loop(0, npw // 16, zero, 0)
    pltpu.sync_copy(zb, deg_sh.at[pl.ds(s * npw, npw)])
    plsc.subcore_barrier()

    q = c * NS + s
    pltpu.sync_copy(col_hbm.at[q], colb)
    pltpu.sync_copy(ew_hbm.at[q], ewb)

    def chunk(j, carry):
        pltpu.sync_copy(ewb.at[pl.ds(j * DEG_EPB, DEG_EPB)],
                        deg_sh.at[colb.at[j]], add=True)
        return carry

    lax.fori_loop(0, DEG_CH, chunk, 0)
    plsc.subcore_barrier()
    pltpu.sync_copy(deg_sh.at[pl.ds(s * npw, npw)], zb)
    pltpu.sync_copy(zb, deg_hbm.at[pl.ds(c * NP + s * npw, npw)])


def _make_scatter(split_features, chunks):
    epm = chunks * EPB

    @functools.partial(
        pl.kernel,
        out_type=jax.ShapeDtypeStruct((NC, NP, 128), jnp.float32),
        mesh=_mesh,
        compiler_params=pltpu.CompilerParams(needs_layout_passes=False),
        scratch_types=[
            pltpu.VMEM((epm,), jnp.int32),
            pltpu.VMEM((chunks, EPB), jnp.int32),
            pltpu.VMEM((epm,), jnp.float32),
            pltpu.VMEM((EPB, 128), jnp.float32),
            pltpu.VMEM((RP_BLK, 128), jnp.float32),
            pltpu.VMEM_SHARED((NP, 128), jnp.float32),
        ],
    )
    def k(xs_hbm, row_hbm, col_hbm, ew_hbm, out_hbm, rowb, colb, ewb, gb, ib, acc):
        c = lax.axis_index("c")
        s = lax.axis_index("s")
        src = xs_hbm.at[c] if split_features else xs_hbm

        def zrow(r, carry):
            for f in range(8):
                ib[r, pl.ds(f * 16, 16)] = jnp.zeros((16,), jnp.float32)
            return carry

        lax.fori_loop(0, RP_BLK, zrow, 0)

        def zinit(i, carry):
            pltpu.sync_copy(ib, acc.at[pl.ds(s * RP_S + i * RP_BLK, RP_BLK)])
            return carry

        lax.fori_loop(0, RP_S // RP_BLK, zinit, 0)
        plsc.subcore_barrier()

        def mega(m, mcarry):
            q = (s if split_features else c * NS + s) * MEGAS + m
            pltpu.sync_copy(row_hbm.at[q], rowb)
            pltpu.sync_copy(col_hbm.at[q], colb)
            pltpu.sync_copy(ew_hbm.at[q], ewb)

            def chunk(j, carry):
                pltpu.sync_copy(src.at[rowb.at[pl.ds(j * EPB, EPB)]], gb)

                def edge(e, ecarry):
                    ev = jnp.full((16,), j * EPB + e, jnp.int32)
                    w = plsc.load_gather(ewb, [ev])
                    for f in range(8):
                        gb[e, pl.ds(f * 16, 16)] = gb[e, pl.ds(f * 16, 16)] * w
                    return ecarry

                lax.fori_loop(0, EPB, edge, 0)
                pltpu.sync_copy(gb, acc.at[colb.at[j]], add=True)
                return carry

            lax.fori_loop(0, chunks, chunk, 0)
            return mcarry

        lax.fori_loop(0, MEGAS, mega, 0)
        plsc.subcore_barrier()

        def drain(i, carry):
            r0 = s * RP_S + i * RP_BLK
            pltpu.sync_copy(acc.at[pl.ds(r0, RP_BLK)], ib)
            pltpu.sync_copy(ib, out_hbm.at[c].at[pl.ds(r0, RP_BLK)])
            return carry

        lax.fori_loop(0, RP_S // RP_BLK, drain, 0)

    return k


_scatter_l1 = _make_scatter(True, 50)
_scatter_l2 = _make_scatter(False, 25)

_BN = 1024


def _tc1_body(x_ref, w1_ref, deg_ref, xs_ref, dinv_ref):
    deg = deg_ref[...]
    dinv = lax.rsqrt(deg[:, 0:1] + deg[:, 1:2] + 1.0)
    dinv_ref[...] = dinv
    xw = jnp.dot(x_ref[...], w1_ref[...], preferred_element_type=jnp.float32)
    xs = xw * dinv
    xs_ref[0] = xs[:, :128]
    xs_ref[1] = xs[:, 128:]


def _tc1(x, W1, degp):
    return pl.pallas_call(
        _tc1_body,
        grid=(NP // _BN,),
        in_specs=[
            pl.BlockSpec((_BN, 128), lambda i: (i, 0)),
            pl.BlockSpec((128, 256), lambda i: (0, 0)),
            pl.BlockSpec((_BN, 2), lambda i: (i, 0)),
        ],
        out_specs=[
            pl.BlockSpec((2, _BN, 128), lambda i: (0, i, 0)),
            pl.BlockSpec((_BN, 1), lambda i: (i, 0)),
        ],
        out_shape=[
            jax.ShapeDtypeStruct((2, NP, 128), jnp.float32),
            jax.ShapeDtypeStruct((NP, 1), jnp.float32),
        ],
    )(x, W1, degp)


def _tc2_body(tmp_ref, xs_ref, dinv_ref, b1_ref, w2_ref, xs2_ref):
    dinv = dinv_ref[...]
    b1 = b1_ref[...]
    w2 = w2_ref[...]
    h_lo = jnp.maximum((tmp_ref[0] + xs_ref[0]) * dinv + b1[:128], 0.0)
    h_hi = jnp.maximum((tmp_ref[1] + xs_ref[1]) * dinv + b1[128:], 0.0)
    xw2 = (jnp.dot(h_lo, w2[:128], preferred_element_type=jnp.float32)
           + jnp.dot(h_hi, w2[128:], preferred_element_type=jnp.float32))
    xs2_ref[...] = xw2 * dinv


def _tc2(tmp, xs1, dinv, b1, W2):
    return pl.pallas_call(
        _tc2_body,
        grid=(NP // _BN,),
        in_specs=[
            pl.BlockSpec((2, _BN, 128), lambda i: (0, i, 0)),
            pl.BlockSpec((2, _BN, 128), lambda i: (0, i, 0)),
            pl.BlockSpec((_BN, 1), lambda i: (i, 0)),
            pl.BlockSpec((256,), lambda i: (0,)),
            pl.BlockSpec((256, 128), lambda i: (0, 0)),
        ],
        out_specs=pl.BlockSpec((_BN, 128), lambda i: (i, 0)),
        out_shape=jax.ShapeDtypeStruct((NP, 128), jnp.float32),
    )(tmp, xs1, dinv, b1, W2)


def _tc3_body(tmp_ref, xs_ref, dinv_ref, b2_ref, wl1_ref, bl1_ref, wl2_ref,
              bl2_ref, o_ref):
    dinv = dinv_ref[...]
    h = jnp.maximum((tmp_ref[0] + tmp_ref[1] + xs_ref[...]) * dinv
                    + b2_ref[...], 0.0)
    t = jnp.maximum(
        jnp.dot(h, wl1_ref[...], preferred_element_type=jnp.float32)
        + bl1_ref[...], 0.0)
    o_ref[...] = jnp.maximum(
        jnp.dot(t, wl2_ref[...], preferred_element_type=jnp.float32)
        + bl2_ref[...], 0.0)


def _tc3(tmp2, xs2, dinv, b2, Wl1, bl1, Wl2, bl2):
    return pl.pallas_call(
        _tc3_body,
        grid=(NP // _BN,),
        in_specs=[
            pl.BlockSpec((2, _BN, 128), lambda i: (0, i, 0)),
            pl.BlockSpec((_BN, 128), lambda i: (i, 0)),
            pl.BlockSpec((_BN, 1), lambda i: (i, 0)),
            pl.BlockSpec((128,), lambda i: (0,)),
            pl.BlockSpec((128, 512), lambda i: (0, 0)),
            pl.BlockSpec((512,), lambda i: (0,)),
            pl.BlockSpec((512, 128), lambda i: (0, 0)),
            pl.BlockSpec((128,), lambda i: (0,)),
        ],
        out_specs=pl.BlockSpec((_BN, 128), lambda i: (i, 0)),
        out_shape=jax.ShapeDtypeStruct((NP, 128), jnp.float32),
    )(tmp2, xs2, dinv, b2, Wl1, bl1, Wl2, bl2)


def kernel(x, edge_index, edge_weight, W1, b1, W2, b2, Wl1, bl1, Wl2, bl2):
    row = edge_index[0]
    col = edge_index[1]
    colD = col.reshape(NC * NS, DEG_CH, DEG_EPB)
    ewD = edge_weight.reshape(NC * NS, DEG_CH * DEG_EPB)
    rowS1 = row.reshape(NS * MEGAS, 50 * EPB)
    colS1 = col.reshape(NS * MEGAS, 50, EPB)
    ewS1 = edge_weight.reshape(NS * MEGAS, 50 * EPB)
    rowS2 = row.reshape(NC * NS * MEGAS, 25 * EPB)
    colS2 = col.reshape(NC * NS * MEGAS, 25, EPB)
    ewS2 = edge_weight.reshape(NC * NS * MEGAS, 25 * EPB)

    degp2 = _deg(colD, ewD)
    degp = degp2.reshape(NC, NP).T

    x_p = jnp.pad(x, ((0, NP - N), (0, 0)))
    xs1, dinv = _tc1(x_p, W1, degp)
    tmp1 = _scatter_l1(xs1, rowS1, colS1, ewS1)
    xs2 = _tc2(tmp1, xs1, dinv, b1, W2)
    tmp2 = _scatter_l2(xs2, rowS2, colS2, ewS2)
    return _tc3(tmp2, xs2, dinv, b2, Wl1, bl1, Wl2, bl2)[:N]


# speedup vs baseline: 18.3545x; 1.6902x over previous
import functools

import jax
import jax.numpy as jnp
from jax import lax
from jax.experimental import pallas as pl
from jax.experimental.pallas import tpu as pltpu
from jax.experimental.pallas import tpu_sc as plsc

N = 10000
E = 320000
NC = 2
NS = 16
NP = 10240

DEG_CH, DEG_EPB = 125, 80
EPB = 80
RP_S = NP // NS
RP_BLK = 64
MEGAS = 5

_mesh = plsc.VectorSubcoreMesh(core_axis_name="c", subcore_axis_name="s")


@functools.partial(
    pl.kernel,
    out_type=jax.ShapeDtypeStruct((NC * NP,), jnp.float32),
    mesh=_mesh,
    compiler_params=pltpu.CompilerParams(needs_layout_passes=False),
    scratch_types=[
        pltpu.VMEM((DEG_CH, DEG_EPB), jnp.int32),
        pltpu.VMEM((DEG_CH * DEG_EPB,), jnp.float32),
        pltpu.VMEM((NP // NS,), jnp.float32),
        pltpu.VMEM_SHARED((NP,), jnp.float32),
    ],
)
def _deg(col_hbm, ew_hbm, deg_hbm, colb, ewb, zb, deg_sh):
    c = lax.axis_index("c")
    s = lax.axis_index("s")
    npw = NP // NS

    def zero(i, carry):
        zb[pl.ds(i * 16, 16)] = jnp.zeros((16,), jnp.float32)
        return carry

    lax.fori_loop(0, npw // 16, zero, 0)
    pltpu.sync_copy(zb, deg_sh.at[pl.ds(s * npw, npw)])
    plsc.subcore_barrier()

    q = c * NS + s
    pltpu.sync_copy(col_hbm.at[q], colb)
    pltpu.sync_copy(ew_hbm.at[q], ewb)

    def chunk(j, carry):
        pltpu.sync_copy(ewb.at[pl.ds(j * DEG_EPB, DEG_EPB)],
                        deg_sh.at[colb.at[j]], add=True)
        return carry

    lax.fori_loop(0, DEG_CH, chunk, 0)
    plsc.subcore_barrier()
    pltpu.sync_copy(deg_sh.at[pl.ds(s * npw, npw)], zb)
    pltpu.sync_copy(zb, deg_hbm.at[pl.ds(c * NP + s * npw, npw)])


def _make_scatter(split_features, chunks):
    epm = chunks * EPB

    @functools.partial(
        pl.kernel,
        out_type=jax.ShapeDtypeStruct((NC, NP, 128), jnp.float32),
        mesh=_mesh,
        compiler_params=pltpu.CompilerParams(needs_layout_passes=False),
        scratch_types=[
            pltpu.VMEM((epm,), jnp.int32),
            pltpu.VMEM((chunks, EPB), jnp.int32),
            pltpu.VMEM((epm,), jnp.float32),
            pltpu.VMEM((EPB, 128), jnp.float32),
            pltpu.VMEM((EPB, 128), jnp.float32),
            pltpu.VMEM((RP_BLK, 128), jnp.float32),
            pltpu.VMEM_SHARED((NP, 128), jnp.float32),
            pltpu.SemaphoreType.DMA,
            pltpu.SemaphoreType.DMA,
            pltpu.SemaphoreType.DMA,
            pltpu.SemaphoreType.DMA,
        ],
    )
    def k(xs_hbm, row_hbm, col_hbm, ew_hbm, out_hbm, rowb, colb, ewb,
          gb0, gb1, ib, acc, gs0, gs1, ss0, ss1):
        c = lax.axis_index("c")
        s = lax.axis_index("s")
        src = xs_hbm.at[c] if split_features else xs_hbm

        def zrow(r, carry):
            for f in range(8):
                ib[r, pl.ds(f * 16, 16)] = jnp.zeros((16,), jnp.float32)
            return carry

        lax.fori_loop(0, RP_BLK, zrow, 0)

        def zinit(i, carry):
            pltpu.sync_copy(ib, acc.at[pl.ds(s * RP_S + i * RP_BLK, RP_BLK)])
            return carry

        lax.fori_loop(0, RP_S // RP_BLK, zinit, 0)
        plsc.subcore_barrier()

        def g_desc(j, gbuf, sem):
            return pltpu.make_async_copy(
                src.at[rowb.at[pl.ds(j * EPB, EPB)]], gbuf, sem)

        def s_desc(j, gbuf, sem):
            return pltpu.make_async_copy(gbuf, acc.at[colb.at[j]], sem)

        def scale(j, gbuf):
            def edge4(e4, ecarry):
                for u in range(4):
                    e = e4 * 4 + u
                    ev = jnp.full((16,), j * EPB + e, jnp.int32)
                    w = plsc.load_gather(ewb, [ev])
                    for f in range(8):
                        gbuf[e, pl.ds(f * 16, 16)] = gbuf[e, pl.ds(f * 16, 16)] * w
                return ecarry

            lax.fori_loop(0, EPB // 4, edge4, 0)

        pairs, tail = chunks // 2, chunks % 2

        def mega(m, mcarry):
            q = (s if split_features else c * NS + s) * MEGAS + m
            pltpu.sync_copy(row_hbm.at[q], rowb)
            pltpu.sync_copy(col_hbm.at[q], colb)
            pltpu.sync_copy(ew_hbm.at[q], ewb)
            g_desc(0, gb0, gs0).start()

            def pair(p, carry):
                j0 = 2 * p
                j1 = j0 + 1
                g_desc(j0, gb0, gs0).wait()

                @pl.when(p > 0)
                def _():
                    s_desc(j1 - 2, gb1, ss1).wait()
                g_desc(j1, gb1, gs1).start()
                scale(j0, gb0)
                s_desc(j0, gb0, ss0).start(add=True)
                g_desc(j1, gb1, gs1).wait()
                s_desc(j0, gb0, ss0).wait()

                @pl.when((p < pairs - 1) | (tail == 1))
                def _():
                    g_desc(j0 + 2, gb0, gs0).start()
                scale(j1, gb1)
                s_desc(j1, gb1, ss1).start(add=True)
                return carry

            lax.fori_loop(0, pairs, pair, 0)
            s_desc(2 * pairs - 1, gb1, ss1).wait()
            if tail:
                jt = chunks - 1
                g_desc(jt, gb0, gs0).wait()
                scale(jt, gb0)
                s_desc(jt, gb0, ss0).start(add=True)
                s_desc(jt, gb0, ss0).wait()
            return mcarry

        lax.fori_loop(0, MEGAS, mega, 0)
        plsc.subcore_barrier()

        def drain(i, carry):
            r0 = s * RP_S + i * RP_BLK
            pltpu.sync_copy(acc.at[pl.ds(r0, RP_BLK)], ib)
            pltpu.sync_copy(ib, out_hbm.at[c].at[pl.ds(r0, RP_BLK)])
            return carry

        lax.fori_loop(0, RP_S // RP_BLK, drain, 0)

    return k


_scatter_l1 = _make_scatter(True, 50)
_scatter_l2 = _make_scatter(False, 25)

_BN = 1024


def _tc1_body(x_ref, w1_ref, deg_ref, xs_ref, dinv_ref):
    deg = deg_ref[...]
    dinv = lax.rsqrt(deg[:, 0:1] + deg[:, 1:2] + 1.0)
    dinv_ref[...] = dinv
    xw = jnp.dot(x_ref[...], w1_ref[...], preferred_element_type=jnp.float32)
    xs = xw * dinv
    xs_ref[0] = xs[:, :128]
    xs_ref[1] = xs[:, 128:]


def _tc1(x, W1, degp):
    return pl.pallas_call(
        _tc1_body,
        grid=(NP // _BN,),
        in_specs=[
            pl.BlockSpec((_BN, 128), lambda i: (i, 0)),
            pl.BlockSpec((128, 256), lambda i: (0, 0)),
            pl.BlockSpec((_BN, 2), lambda i: (i, 0)),
        ],
        out_specs=[
            pl.BlockSpec((2, _BN, 128), lambda i: (0, i, 0)),
            pl.BlockSpec((_BN, 1), lambda i: (i, 0)),
        ],
        out_shape=[
            jax.ShapeDtypeStruct((2, NP, 128), jnp.float32),
            jax.ShapeDtypeStruct((NP, 1), jnp.float32),
        ],
    )(x, W1, degp)


def _tc2_body(tmp_ref, xs_ref, dinv_ref, b1_ref, w2_ref, xs2_ref):
    dinv = dinv_ref[...]
    b1 = b1_ref[...]
    w2 = w2_ref[...]
    h_lo = jnp.maximum((tmp_ref[0] + xs_ref[0]) * dinv + b1[:128], 0.0)
    h_hi = jnp.maximum((tmp_ref[1] + xs_ref[1]) * dinv + b1[128:], 0.0)
    xw2 = (jnp.dot(h_lo, w2[:128], preferred_element_type=jnp.float32)
           + jnp.dot(h_hi, w2[128:], preferred_element_type=jnp.float32))
    xs2_ref[...] = xw2 * dinv


def _tc2(tmp, xs1, dinv, b1, W2):
    return pl.pallas_call(
        _tc2_body,
        grid=(NP // _BN,),
        in_specs=[
            pl.BlockSpec((2, _BN, 128), lambda i: (0, i, 0)),
            pl.BlockSpec((2, _BN, 128), lambda i: (0, i, 0)),
            pl.BlockSpec((_BN, 1), lambda i: (i, 0)),
            pl.BlockSpec((256,), lambda i: (0,)),
            pl.BlockSpec((256, 128), lambda i: (0, 0)),
        ],
        out_specs=pl.BlockSpec((_BN, 128), lambda i: (i, 0)),
        out_shape=jax.ShapeDtypeStruct((NP, 128), jnp.float32),
    )(tmp, xs1, dinv, b1, W2)


def _tc3_body(tmp_ref, xs_ref, dinv_ref, b2_ref, wl1_ref, bl1_ref, wl2_ref,
              bl2_ref, o_ref):
    dinv = dinv_ref[...]
    h = jnp.maximum((tmp_ref[0] + tmp_ref[1] + xs_ref[...]) * dinv
                    + b2_ref[...], 0.0)
    t = jnp.maximum(
        jnp.dot(h, wl1_ref[...], preferred_element_type=jnp.float32)
        + bl1_ref[...], 0.0)
    o_ref[...] = jnp.maximum(
        jnp.dot(t, wl2_ref[...], preferred_element_type=jnp.float32)
        + bl2_ref[...], 0.0)


def _tc3(tmp2, xs2, dinv, b2, Wl1, bl1, Wl2, bl2):
    return pl.pallas_call(
        _tc3_body,
        grid=(NP // _BN,),
        in_specs=[
            pl.BlockSpec((2, _BN, 128), lambda i: (0, i, 0)),
            pl.BlockSpec((_BN, 128), lambda i: (i, 0)),
            pl.BlockSpec((_BN, 1), lambda i: (i, 0)),
            pl.BlockSpec((128,), lambda i: (0,)),
            pl.BlockSpec((128, 512), lambda i: (0, 0)),
            pl.BlockSpec((512,), lambda i: (0,)),
            pl.BlockSpec((512, 128), lambda i: (0, 0)),
            pl.BlockSpec((128,), lambda i: (0,)),
        ],
        out_specs=pl.BlockSpec((_BN, 128), lambda i: (i, 0)),
        out_shape=jax.ShapeDtypeStruct((NP, 128), jnp.float32),
    )(tmp2, xs2, dinv, b2, Wl1, bl1, Wl2, bl2)


def kernel(x, edge_index, edge_weight, W1, b1, W2, b2, Wl1, bl1, Wl2, bl2):
    row = edge_index[0]
    col = edge_index[1]
    colD = col.reshape(NC * NS, DEG_CH, DEG_EPB)
    ewD = edge_weight.reshape(NC * NS, DEG_CH * DEG_EPB)
    rowS1 = row.reshape(NS * MEGAS, 50 * EPB)
    colS1 = col.reshape(NS * MEGAS, 50, EPB)
    ewS1 = edge_weight.reshape(NS * MEGAS, 50 * EPB)
    rowS2 = row.reshape(NC * NS * MEGAS, 25 * EPB)
    colS2 = col.reshape(NC * NS * MEGAS, 25, EPB)
    ewS2 = edge_weight.reshape(NC * NS * MEGAS, 25 * EPB)

    degp2 = _deg(colD, ewD)
    degp = degp2.reshape(NC, NP).T

    x_p = jnp.pad(x, ((0, NP - N), (0, 0)))
    xs1, dinv = _tc1(x_p, W1, degp)
    tmp1 = _scatter_l1(xs1, rowS1, colS1, ewS1)
    xs2 = _tc2(tmp1, xs1, dinv, b1, W2)
    tmp2 = _scatter_l2(xs2, rowS2, colS2, ewS2)
    return _tc3(tmp2, xs2, dinv, b2, Wl1, bl1, Wl2, bl2)[:N]


# speedup vs baseline: 19.4204x; 1.0581x over previous
import functools

import jax
import jax.numpy as jnp
from jax import lax
from jax.experimental import pallas as pl
from jax.experimental.pallas import tpu as pltpu
from jax.experimental.pallas import tpu_sc as plsc

N = 10000
E = 320000
NC = 2
NS = 16
NP = 10240

DEG_CH, DEG_EPB = 125, 80
EPB = 100
MCH = 10
RP_S = NP // NS
RP_BLK = 64

_mesh = plsc.VectorSubcoreMesh(core_axis_name="c", subcore_axis_name="s")


@functools.partial(
    pl.kernel,
    out_type=jax.ShapeDtypeStruct((NC * NP,), jnp.float32),
    mesh=_mesh,
    compiler_params=pltpu.CompilerParams(needs_layout_passes=False),
    scratch_types=[
        pltpu.VMEM((DEG_CH, DEG_EPB), jnp.int32),
        pltpu.VMEM((DEG_CH * DEG_EPB,), jnp.float32),
        pltpu.VMEM((NP // NS,), jnp.float32),
        pltpu.VMEM_SHARED((NP,), jnp.float32),
    ],
)
def _deg(col_hbm, ew_hbm, deg_hbm, colb, ewb, zb, deg_sh):
    c = lax.axis_index("c")
    s = lax.axis_index("s")
    npw = NP // NS

    def zero(i, carry):
        zb[pl.ds(i * 16, 16)] = jnp.zeros((16,), jnp.float32)
        return carry

    lax.fori_loop(0, npw // 16, zero, 0)
    pltpu.sync_copy(zb, deg_sh.at[pl.ds(s * npw, npw)])
    plsc.subcore_barrier()

    q = c * NS + s
    pltpu.sync_copy(col_hbm.at[q], colb)
    pltpu.sync_copy(ew_hbm.at[q], ewb)

    def chunk(j, carry):
        pltpu.sync_copy(ewb.at[pl.ds(j * DEG_EPB, DEG_EPB)],
                        deg_sh.at[colb.at[j]], add=True)
        return carry

    lax.fori_loop(0, DEG_CH, chunk, 0)
    plsc.subcore_barrier()
    pltpu.sync_copy(deg_sh.at[pl.ds(s * npw, npw)], zb)
    pltpu.sync_copy(zb, deg_hbm.at[pl.ds(c * NP + s * npw, npw)])


def _make_scatter(split_features, nmega):
    epm = MCH * EPB
    pairs = MCH // 2

    @functools.partial(
        pl.kernel,
        out_type=jax.ShapeDtypeStruct((NC, NP, 128), jnp.float32),
        mesh=_mesh,
        compiler_params=pltpu.CompilerParams(needs_layout_passes=False),
        scratch_types=[
            pltpu.VMEM((MCH, EPB), jnp.int32),
            pltpu.VMEM((MCH, EPB), jnp.int32),
            pltpu.VMEM((epm,), jnp.float32),
            pltpu.VMEM((MCH, EPB), jnp.int32),
            pltpu.VMEM((MCH, EPB), jnp.int32),
            pltpu.VMEM((epm,), jnp.float32),
            pltpu.VMEM((EPB, 128), jnp.float32),
            pltpu.VMEM((EPB, 128), jnp.float32),
            pltpu.VMEM((RP_BLK, 128), jnp.float32),
            pltpu.VMEM_SHARED((NP, 128), jnp.float32),
            pltpu.SemaphoreType.DMA,
            pltpu.SemaphoreType.DMA,
            pltpu.SemaphoreType.DMA,
            pltpu.SemaphoreType.DMA,
            pltpu.SemaphoreType.DMA,
        ],
    )
    def k(xs_hbm, row_hbm, col_hbm, ew_hbm, out_hbm,
          rowa, cola, ewa, rowq, colq, ewq, gb0, gb1, ib, acc,
          gs0, gs1, ss0, ss1, ps):
        c = lax.axis_index("c")
        s = lax.axis_index("s")
        src = xs_hbm.at[c] if split_features else xs_hbm
        qbase = (s if split_features else c * NS + s) * nmega

        def zrow(r, carry):
            for f in range(8):
                ib[r, pl.ds(f * 16, 16)] = jnp.zeros((16,), jnp.float32)
            return carry

        lax.fori_loop(0, RP_BLK, zrow, 0)

        def zinit(i, carry):
            pltpu.sync_copy(ib, acc.at[pl.ds(s * RP_S + i * RP_BLK, RP_BLK)])
            return carry

        lax.fori_loop(0, RP_S // RP_BLK, zinit, 0)
        plsc.subcore_barrier()

        def idx_descs(q, rb, cb, eb):
            return [pltpu.make_async_copy(row_hbm.at[q], rb, ps),
                    pltpu.make_async_copy(col_hbm.at[q], cb, ps),
                    pltpu.make_async_copy(ew_hbm.at[q], eb, ps)]

        def g_desc(j, rb, gbuf, sem):
            return pltpu.make_async_copy(src.at[rb.at[j]], gbuf, sem)

        def s_desc(j, cb, gbuf, sem):
            return pltpu.make_async_copy(gbuf, acc.at[cb.at[j]], sem)

        def scale(j, eb, gbuf):
            def edge4(e4, ecarry):
                for u in range(4):
                    e = e4 * 4 + u
                    ev = jnp.full((16,), j * EPB + e, jnp.int32)
                    w = plsc.load_gather(eb, [ev])
                    for f in range(8):
                        gbuf[e, pl.ds(f * 16, 16)] = gbuf[e, pl.ds(f * 16, 16)] * w
                return ecarry

            lax.fori_loop(0, EPB // 4, edge4, 0)

        def process_mega(bufs, nbufs, qn, cross):
            rb, cb, eb = bufs
            nrb, _, _ = nbufs

            def pair(p, carry):
                j0 = 2 * p
                j1 = j0 + 1
                g_desc(j0, rb, gb0, gs0).wait()

                @pl.when(p > 0)
                def _():
                    s_desc(j1 - 2, cb, gb1, ss1).wait()
                g_desc(j1, rb, gb1, gs1).start()
                scale(j0, eb, gb0)
                s_desc(j0, cb, gb0, ss0).start(add=True)
                g_desc(j1, rb, gb1, gs1).wait()
                s_desc(j0, cb, gb0, ss0).wait()

                @pl.when(p < pairs - 1)
                def _():
                    g_desc(j0 + 2, rb, gb0, gs0).start()

                @pl.when((p == pairs - 1) & cross)
                def _():
                    for d in idx_descs(qn, *nbufs):
                        d.wait()
                    g_desc(0, nrb, gb0, gs0).start()
                scale(j1, eb, gb1)
                s_desc(j1, cb, gb1, ss1).start(add=True)
                return carry

            lax.fori_loop(0, pairs, pair, 0)
            s_desc(MCH - 1, cb, gb1, ss1).wait()

        A = (rowa, cola, ewa)
        B = (rowq, colq, ewq)
        pltpu.sync_copy(row_hbm.at[qbase], rowa)
        pltpu.sync_copy(col_hbm.at[qbase], cola)
        pltpu.sync_copy(ew_hbm.at[qbase], ewa)
        for d in idx_descs(qbase + 1, *B):
            d.start()
        g_desc(0, rowa, gb0, gs0).start()

        true_ = jnp.bool_(True)

        def megapair(i, carry):
            m0 = 2 * i
            process_mega(A, B, qbase + m0 + 1, true_)

            @pl.when(m0 + 2 < nmega)
            def _():
                for d in idx_descs(qbase + m0 + 2, *A):
                    d.start()
            process_mega(B, A, qbase + m0 + 2, m0 + 2 < nmega)

            @pl.when(m0 + 3 < nmega)
            def _():
                for d in idx_descs(qbase + m0 + 3, *B):
                    d.start()
            return carry

        lax.fori_loop(0, nmega // 2, megapair, 0)
        plsc.subcore_barrier()

        def drain(i, carry):
            r0 = s * RP_S + i * RP_BLK
            pltpu.sync_copy(acc.at[pl.ds(r0, RP_BLK)], ib)
            pltpu.sync_copy(ib, out_hbm.at[c].at[pl.ds(r0, RP_BLK)])
            return carry

        lax.fori_loop(0, RP_S // RP_BLK, drain, 0)

    return k


_scatter_l1 = _make_scatter(True, 20)
_scatter_l2 = _make_scatter(False, 10)

_BN = 1024


def _tc1_body(x_ref, w1_ref, deg_ref, xs_ref, dinv_ref):
    deg = deg_ref[...]
    dinv = lax.rsqrt(deg[:, 0:1] + deg[:, 1:2] + 1.0)
    dinv_ref[...] = dinv
    xw = jnp.dot(x_ref[...], w1_ref[...], preferred_element_type=jnp.float32)
    xs = xw * dinv
    xs_ref[0] = xs[:, :128]
    xs_ref[1] = xs[:, 128:]


def _tc1(x, W1, degp):
    return pl.pallas_call(
        _tc1_body,
        grid=(NP // _BN,),
        in_specs=[
            pl.BlockSpec((_BN, 128), lambda i: (i, 0)),
            pl.BlockSpec((128, 256), lambda i: (0, 0)),
            pl.BlockSpec((_BN, 2), lambda i: (i, 0)),
        ],
        out_specs=[
            pl.BlockSpec((2, _BN, 128), lambda i: (0, i, 0)),
            pl.BlockSpec((_BN, 1), lambda i: (i, 0)),
        ],
        out_shape=[
            jax.ShapeDtypeStruct((2, NP, 128), jnp.float32),
            jax.ShapeDtypeStruct((NP, 1), jnp.float32),
        ],
    )(x, W1, degp)


def _tc2_body(tmp_ref, xs_ref, dinv_ref, b1_ref, w2_ref, xs2_ref):
    dinv = dinv_ref[...]
    b1 = b1_ref[...]
    w2 = w2_ref[...]
    h_lo = jnp.maximum((tmp_ref[0] + xs_ref[0]) * dinv + b1[:128], 0.0)
    h_hi = jnp.maximum((tmp_ref[1] + xs_ref[1]) * dinv + b1[128:], 0.0)
    xw2 = (jnp.dot(h_lo, w2[:128], preferred_element_type=jnp.float32)
           + jnp.dot(h_hi, w2[128:], preferred_element_type=jnp.float32))
    xs2_ref[...] = xw2 * dinv


def _tc2(tmp, xs1, dinv, b1, W2):
    return pl.pallas_call(
        _tc2_body,
        grid=(NP // _BN,),
        in_specs=[
            pl.BlockSpec((2, _BN, 128), lambda i: (0, i, 0)),
            pl.BlockSpec((2, _BN, 128), lambda i: (0, i, 0)),
            pl.BlockSpec((_BN, 1), lambda i: (i, 0)),
            pl.BlockSpec((256,), lambda i: (0,)),
            pl.BlockSpec((256, 128), lambda i: (0, 0)),
        ],
        out_specs=pl.BlockSpec((_BN, 128), lambda i: (i, 0)),
        out_shape=jax.ShapeDtypeStruct((NP, 128), jnp.float32),
    )(tmp, xs1, dinv, b1, W2)


def _tc3_body(tmp_ref, xs_ref, dinv_ref, b2_ref, wl1_ref, bl1_ref, wl2_ref,
              bl2_ref, o_ref):
    dinv = dinv_ref[...]
    h = jnp.maximum((tmp_ref[0] + tmp_ref[1] + xs_ref[...]) * dinv
                    + b2_ref[...], 0.0)
    t = jnp.maximum(
        jnp.dot(h, wl1_ref[...], preferred_element_type=jnp.float32)
        + bl1_ref[...], 0.0)
    o_ref[...] = jnp.maximum(
        jnp.dot(t, wl2_ref[...], preferred_element_type=jnp.float32)
        + bl2_ref[...], 0.0)


def _tc3(tmp2, xs2, dinv, b2, Wl1, bl1, Wl2, bl2):
    return pl.pallas_call(
        _tc3_body,
        grid=(NP // _BN,),
        in_specs=[
            pl.BlockSpec((2, _BN, 128), lambda i: (0, i, 0)),
            pl.BlockSpec((_BN, 128), lambda i: (i, 0)),
            pl.BlockSpec((_BN, 1), lambda i: (i, 0)),
            pl.BlockSpec((128,), lambda i: (0,)),
            pl.BlockSpec((128, 512), lambda i: (0, 0)),
            pl.BlockSpec((512,), lambda i: (0,)),
            pl.BlockSpec((512, 128), lambda i: (0, 0)),
            pl.BlockSpec((128,), lambda i: (0,)),
        ],
        out_specs=pl.BlockSpec((_BN, 128), lambda i: (i, 0)),
        out_shape=jax.ShapeDtypeStruct((NP, 128), jnp.float32),
    )(tmp2, xs2, dinv, b2, Wl1, bl1, Wl2, bl2)


def kernel(x, edge_index, edge_weight, W1, b1, W2, b2, Wl1, bl1, Wl2, bl2):
    row = edge_index[0]
    col = edge_index[1]
    colD = col.reshape(NC * NS, DEG_CH, DEG_EPB)
    ewD = edge_weight.reshape(NC * NS, DEG_CH * DEG_EPB)
    rowS = row.reshape(E // (MCH * EPB), MCH, EPB)
    colS = col.reshape(E // (MCH * EPB), MCH, EPB)
    ewS = edge_weight.reshape(E // (MCH * EPB), MCH * EPB)

    degp2 = _deg(colD, ewD)
    degp = degp2.reshape(NC, NP).T

    x_p = jnp.pad(x, ((0, NP - N), (0, 0)))
    xs1, dinv = _tc1(x_p, W1, degp)
    tmp1 = _scatter_l1(xs1, rowS, colS, ewS)
    xs2 = _tc2(tmp1, xs1, dinv, b1, W2)
    tmp2 = _scatter_l2(xs2, rowS, colS, ewS)
    return _tc3(tmp2, xs2, dinv, b2, Wl1, bl1, Wl2, bl2)[:N]


# speedup vs baseline: 19.7063x; 1.0147x over previous
import functools

import jax
import jax.numpy as jnp
from jax import lax
from jax.experimental import pallas as pl
from jax.experimental.pallas import tpu as pltpu
from jax.experimental.pallas import tpu_sc as plsc

N = 10000
E = 320000
NC = 2
NS = 16
NP = 10240

DEG_CH, DEG_EPB = 125, 80
EPB = 100
MCH = 10
RP_S = NP // NS
RP_BLK = 64

_mesh = plsc.VectorSubcoreMesh(core_axis_name="c", subcore_axis_name="s")


@functools.partial(
    pl.kernel,
    out_type=jax.ShapeDtypeStruct((NC * NP,), jnp.float32),
    mesh=_mesh,
    compiler_params=pltpu.CompilerParams(needs_layout_passes=False),
    scratch_types=[
        pltpu.VMEM((DEG_CH, DEG_EPB), jnp.int32),
        pltpu.VMEM((DEG_CH * DEG_EPB,), jnp.float32),
        pltpu.VMEM((NP // NS,), jnp.float32),
        pltpu.VMEM_SHARED((NP,), jnp.float32),
    ],
)
def _deg(col_hbm, ew_hbm, deg_hbm, colb, ewb, zb, deg_sh):
    c = lax.axis_index("c")
    s = lax.axis_index("s")
    npw = NP // NS

    def zero(i, carry):
        zb[pl.ds(i * 16, 16)] = jnp.zeros((16,), jnp.float32)
        return carry

    lax.fori_loop(0, npw // 16, zero, 0)
    pltpu.sync_copy(zb, deg_sh.at[pl.ds(s * npw, npw)])
    plsc.subcore_barrier()

    q = c * NS + s
    pltpu.sync_copy(col_hbm.at[q], colb)
    pltpu.sync_copy(ew_hbm.at[q], ewb)

    def chunk(j, carry):
        pltpu.sync_copy(ewb.at[pl.ds(j * DEG_EPB, DEG_EPB)],
                        deg_sh.at[colb.at[j]], add=True)
        return carry

    lax.fori_loop(0, DEG_CH, chunk, 0)
    plsc.subcore_barrier()
    pltpu.sync_copy(deg_sh.at[pl.ds(s * npw, npw)], zb)
    pltpu.sync_copy(zb, deg_hbm.at[pl.ds(c * NP + s * npw, npw)])


def _make_scatter(split_features, nmega):
    epm = MCH * EPB
    pairs = MCH // 2

    @functools.partial(
        pl.kernel,
        out_type=jax.ShapeDtypeStruct((NC, NP, 128), jnp.float32),
        mesh=_mesh,
        compiler_params=pltpu.CompilerParams(needs_layout_passes=False),
        scratch_types=[
            pltpu.VMEM((MCH, EPB), jnp.int32),
            pltpu.VMEM((MCH, EPB), jnp.int32),
            pltpu.VMEM((epm,), jnp.float32),
            pltpu.VMEM((MCH, EPB), jnp.int32),
            pltpu.VMEM((MCH, EPB), jnp.int32),
            pltpu.VMEM((epm,), jnp.float32),
            pltpu.VMEM((EPB, 128), jnp.float32),
            pltpu.VMEM((EPB, 128), jnp.float32),
            pltpu.VMEM((RP_BLK, 128), jnp.float32),
            pltpu.VMEM_SHARED((NP, 128), jnp.float32),
            pltpu.SemaphoreType.DMA,
            pltpu.SemaphoreType.DMA,
            pltpu.SemaphoreType.DMA,
            pltpu.SemaphoreType.DMA,
            pltpu.SemaphoreType.DMA,
        ],
    )
    def k(xs_hbm, row_hbm, col_hbm, ew_hbm, out_hbm,
          rowa, cola, ewa, rowq, colq, ewq, gb0, gb1, ib, acc,
          gs0, gs1, ss0, ss1, ps):
        c = lax.axis_index("c")
        s = lax.axis_index("s")
        src = xs_hbm.at[c] if split_features else xs_hbm
        qbase = (s if split_features else c * NS + s) * nmega

        def zrow(r, carry):
            for f in range(8):
                ib[r, pl.ds(f * 16, 16)] = jnp.zeros((16,), jnp.float32)
            return carry

        lax.fori_loop(0, RP_BLK, zrow, 0)

        def zinit(i, carry):
            pltpu.sync_copy(ib, acc.at[pl.ds(s * RP_S + i * RP_BLK, RP_BLK)])
            return carry

        lax.fori_loop(0, RP_S // RP_BLK, zinit, 0)
        plsc.subcore_barrier()

        def idx_descs(q, rb, cb, eb):
            return [pltpu.make_async_copy(row_hbm.at[q], rb, ps),
                    pltpu.make_async_copy(col_hbm.at[q], cb, ps),
                    pltpu.make_async_copy(ew_hbm.at[q], eb, ps)]

        def g_desc(j, rb, gbuf, sem):
            return pltpu.make_async_copy(src.at[rb.at[j]], gbuf, sem)

        def s_desc(j, cb, gbuf, sem):
            return pltpu.make_async_copy(gbuf, acc.at[cb.at[j]], sem)

        def scale(j, eb, gbuf):
            def edge4(e4, ecarry):
                for u in range(4):
                    e = e4 * 4 + u
                    ev = jnp.full((16,), j * EPB + e, jnp.int32)
                    w = plsc.load_gather(eb, [ev])
                    for f in range(8):
                        gbuf[e, pl.ds(f * 16, 16)] = gbuf[e, pl.ds(f * 16, 16)] * w
                return ecarry

            lax.fori_loop(0, EPB // 4, edge4, 0)

        def process_mega(bufs, nbufs, qn, cross):
            rb, cb, eb = bufs
            nrb, _, _ = nbufs

            def pair(p, carry):
                j0 = 2 * p
                j1 = j0 + 1
                g_desc(j0, rb, gb0, gs0).wait()

                @pl.when(p > 0)
                def _():
                    s_desc(j1 - 2, cb, gb1, ss1).wait()
                g_desc(j1, rb, gb1, gs1).start()
                scale(j0, eb, gb0)
                s_desc(j0, cb, gb0, ss0).start(add=True)
                g_desc(j1, rb, gb1, gs1).wait()
                s_desc(j0, cb, gb0, ss0).wait()

                @pl.when(p < pairs - 1)
                def _():
                    g_desc(j0 + 2, rb, gb0, gs0).start()

                @pl.when((p == pairs - 1) & cross)
                def _():
                    for d in idx_descs(qn, *nbufs):
                        d.wait()
                    g_desc(0, nrb, gb0, gs0).start()
                scale(j1, eb, gb1)
                s_desc(j1, cb, gb1, ss1).start(add=True)
                return carry

            lax.fori_loop(0, pairs, pair, 0)
            s_desc(MCH - 1, cb, gb1, ss1).wait()

        A = (rowa, cola, ewa)
        B = (rowq, colq, ewq)
        pltpu.sync_copy(row_hbm.at[qbase], rowa)
        pltpu.sync_copy(col_hbm.at[qbase], cola)
        pltpu.sync_copy(ew_hbm.at[qbase], ewa)
        for d in idx_descs(qbase + 1, *B):
            d.start()
        g_desc(0, rowa, gb0, gs0).start()

        true_ = jnp.bool_(True)

        def megapair(i, carry):
            m0 = 2 * i
            process_mega(A, B, qbase + m0 + 1, true_)

            @pl.when(m0 + 2 < nmega)
            def _():
                for d in idx_descs(qbase + m0 + 2, *A):
                    d.start()
            process_mega(B, A, qbase + m0 + 2, m0 + 2 < nmega)

            @pl.when(m0 + 3 < nmega)
            def _():
                for d in idx_descs(qbase + m0 + 3, *B):
                    d.start()
            return carry

        lax.fori_loop(0, nmega // 2, megapair, 0)
        plsc.subcore_barrier()

        def drain(i, carry):
            r0 = s * RP_S + i * RP_BLK
            pltpu.sync_copy(acc.at[pl.ds(r0, RP_BLK)],
                            out_hbm.at[c].at[pl.ds(r0, RP_BLK)])
            return carry

        lax.fori_loop(0, RP_S // RP_BLK, drain, 0)

    return k


_scatter_l1 = _make_scatter(True, 20)
_scatter_l2 = _make_scatter(False, 10)

_BN = 1024


def _tc1_body(x_ref, w1_ref, deg_ref, xs_ref, dinv_ref):
    deg = deg_ref[...]
    dinv = lax.rsqrt((deg[0:1, :] + deg[1:2, :]).T + 1.0)
    dinv_ref[...] = dinv
    xw = jnp.dot(x_ref[...], w1_ref[...], preferred_element_type=jnp.float32)
    xs = xw * dinv
    xs_ref[0] = xs[:, :128]
    xs_ref[1] = xs[:, 128:]


def _tc1(x, W1, degp):
    return pl.pallas_call(
        _tc1_body,
        grid=(NP // _BN,),
        in_specs=[
            pl.BlockSpec((_BN, 128), lambda i: (i, 0)),
            pl.BlockSpec((128, 256), lambda i: (0, 0)),
            pl.BlockSpec((2, _BN), lambda i: (0, i)),
        ],
        out_specs=[
            pl.BlockSpec((2, _BN, 128), lambda i: (0, i, 0)),
            pl.BlockSpec((_BN, 1), lambda i: (i, 0)),
        ],
        out_shape=[
            jax.ShapeDtypeStruct((2, NP, 128), jnp.float32),
            jax.ShapeDtypeStruct((NP, 1), jnp.float32),
        ],
    )(x, W1, degp)


def _tc2_body(tmp_ref, xs_ref, dinv_ref, b1_ref, w2_ref, xs2_ref):
    dinv = dinv_ref[...]
    b1 = b1_ref[...]
    w2 = w2_ref[...]
    h_lo = jnp.maximum((tmp_ref[0] + xs_ref[0]) * dinv + b1[:128], 0.0)
    h_hi = jnp.maximum((tmp_ref[1] + xs_ref[1]) * dinv + b1[128:], 0.0)
    xw2 = (jnp.dot(h_lo, w2[:128], preferred_element_type=jnp.float32)
           + jnp.dot(h_hi, w2[128:], preferred_element_type=jnp.float32))
    xs2_ref[...] = xw2 * dinv


def _tc2(tmp, xs1, dinv, b1, W2):
    return pl.pallas_call(
        _tc2_body,
        grid=(NP // _BN,),
        in_specs=[
            pl.BlockSpec((2, _BN, 128), lambda i: (0, i, 0)),
            pl.BlockSpec((2, _BN, 128), lambda i: (0, i, 0)),
            pl.BlockSpec((_BN, 1), lambda i: (i, 0)),
            pl.BlockSpec((256,), lambda i: (0,)),
            pl.BlockSpec((256, 128), lambda i: (0, 0)),
        ],
        out_specs=pl.BlockSpec((_BN, 128), lambda i: (i, 0)),
        out_shape=jax.ShapeDtypeStruct((NP, 128), jnp.float32),
    )(tmp, xs1, dinv, b1, W2)


def _tc3_body(tmp_ref, xs_ref, dinv_ref, b2_ref, wl1_ref, bl1_ref, wl2_ref,
              bl2_ref, o_ref):
    dinv = dinv_ref[...]
    h = jnp.maximum((tmp_ref[0] + tmp_ref[1] + xs_ref[...]) * dinv
                    + b2_ref[...], 0.0)
    t = jnp.maximum(
        jnp.dot(h, wl1_ref[...], preferred_element_type=jnp.float32)
        + bl1_ref[...], 0.0)
    o_ref[...] = jnp.maximum(
        jnp.dot(t, wl2_ref[...], preferred_element_type=jnp.float32)
        + bl2_ref[...], 0.0)


def _tc3(tmp2, xs2, dinv, b2, Wl1, bl1, Wl2, bl2):
    return pl.pallas_call(
        _tc3_body,
        grid=(NP // _BN,),
        in_specs=[
            pl.BlockSpec((2, _BN, 128), lambda i: (0, i, 0)),
            pl.BlockSpec((_BN, 128), lambda i: (i, 0)),
            pl.BlockSpec((_BN, 1), lambda i: (i, 0)),
            pl.BlockSpec((128,), lambda i: (0,)),
            pl.BlockSpec((128, 512), lambda i: (0, 0)),
            pl.BlockSpec((512,), lambda i: (0,)),
            pl.BlockSpec((512, 128), lambda i: (0, 0)),
            pl.BlockSpec((128,), lambda i: (0,)),
        ],
        out_specs=pl.BlockSpec((_BN, 128), lambda i: (i, 0)),
        out_shape=jax.ShapeDtypeStruct((N, 128), jnp.float32),
    )(tmp2, xs2, dinv, b2, Wl1, bl1, Wl2, bl2)


def kernel(x, edge_index, edge_weight, W1, b1, W2, b2, Wl1, bl1, Wl2, bl2):
    row = edge_index[0]
    col = edge_index[1]
    colD = col.reshape(NC * NS, DEG_CH, DEG_EPB)
    ewD = edge_weight.reshape(NC * NS, DEG_CH * DEG_EPB)
    rowS = row.reshape(E // (MCH * EPB), MCH, EPB)
    colS = col.reshape(E // (MCH * EPB), MCH, EPB)
    ewS = edge_weight.reshape(E // (MCH * EPB), MCH * EPB)

    degp2 = _deg(colD, ewD)
    degp = degp2.reshape(NC, NP)

    x_p = jnp.pad(x, ((0, NP - N), (0, 0)))
    xs1, dinv = _tc1(x_p, W1, degp)
    tmp1 = _scatter_l1(xs1, rowS, colS, ewS)
    xs2 = _tc2(tmp1, xs1, dinv, b1, W2)
    tmp2 = _scatter_l2(xs2, rowS, colS, ewS)
    return _tc3(tmp2, xs2, dinv, b2, Wl1, bl1, Wl2, bl2)


# speedup vs baseline: 19.8255x; 1.0060x over previous
import functools

import jax
import jax.numpy as jnp
from jax import lax
from jax.experimental import pallas as pl
from jax.experimental.pallas import tpu as pltpu
from jax.experimental.pallas import tpu_sc as plsc

N = 10000
E = 320000
NC = 2
NS = 16
NP = 10240

DEG_CH, DEG_EPB = 125, 80
EPB = 100
MCH = 10
RP_S = NP // NS
RP_BLK = 64

_mesh = plsc.VectorSubcoreMesh(core_axis_name="c", subcore_axis_name="s")


@functools.partial(
    pl.kernel,
    out_type=jax.ShapeDtypeStruct((NC * NP,), jnp.float32),
    mesh=_mesh,
    compiler_params=pltpu.CompilerParams(needs_layout_passes=False),
    scratch_types=[
        pltpu.VMEM((DEG_CH, DEG_EPB), jnp.int32),
        pltpu.VMEM((DEG_CH * DEG_EPB,), jnp.float32),
        pltpu.VMEM((NP // NS,), jnp.float32),
        pltpu.VMEM_SHARED((NP,), jnp.float32),
    ],
)
def _deg(col_hbm, ew_hbm, deg_hbm, colb, ewb, zb, deg_sh):
    c = lax.axis_index("c")
    s = lax.axis_index("s")
    npw = NP // NS

    def zero(i, carry):
        zb[pl.ds(i * 16, 16)] = jnp.zeros((16,), jnp.float32)
        return carry

    lax.fori_loop(0, npw // 16, zero, 0)
    pltpu.sync_copy(zb, deg_sh.at[pl.ds(s * npw, npw)])
    plsc.subcore_barrier()

    q = c * NS + s
    pltpu.sync_copy(col_hbm.at[q], colb)
    pltpu.sync_copy(ew_hbm.at[q], ewb)

    def chunk(j, carry):
        pltpu.sync_copy(ewb.at[pl.ds(j * DEG_EPB, DEG_EPB)],
                        deg_sh.at[colb.at[j]], add=True)
        return carry

    lax.fori_loop(0, DEG_CH, chunk, 0)
    plsc.subcore_barrier()
    pltpu.sync_copy(deg_sh.at[pl.ds(s * npw, npw)], zb)
    pltpu.sync_copy(zb, deg_hbm.at[pl.ds(c * NP + s * npw, npw)])


def _make_scatter(split_features, nmega):
    epm = MCH * EPB
    pairs = MCH // 2

    @functools.partial(
        pl.kernel,
        out_type=jax.ShapeDtypeStruct((NC, NP, 128), jnp.float32),
        mesh=_mesh,
        compiler_params=pltpu.CompilerParams(needs_layout_passes=False),
        scratch_types=[
            pltpu.VMEM((MCH, EPB), jnp.int32),
            pltpu.VMEM((MCH, EPB), jnp.int32),
            pltpu.VMEM((epm,), jnp.float32),
            pltpu.VMEM((MCH, EPB), jnp.int32),
            pltpu.VMEM((MCH, EPB), jnp.int32),
            pltpu.VMEM((epm,), jnp.float32),
            pltpu.VMEM((EPB, 128), jnp.float32),
            pltpu.VMEM((EPB, 128), jnp.float32),
            pltpu.VMEM((RP_BLK, 128), jnp.float32),
            pltpu.VMEM_SHARED((NP, 128), jnp.float32),
            pltpu.SemaphoreType.DMA,
            pltpu.SemaphoreType.DMA,
            pltpu.SemaphoreType.DMA,
            pltpu.SemaphoreType.DMA,
            pltpu.SemaphoreType.DMA,
        ],
    )
    def k(xs_hbm, row_hbm, col_hbm, ew_hbm, out_hbm,
          rowa, cola, ewa, rowq, colq, ewq, gb0, gb1, ib, acc,
          gs0, gs1, ss0, ss1, ps):
        c = lax.axis_index("c")
        s = lax.axis_index("s")
        src = xs_hbm.at[c] if split_features else xs_hbm
        qbase = (s if split_features else c * NS + s) * nmega

        def idx_descs(q, rb, cb, eb):
            return [pltpu.make_async_copy(row_hbm.at[q], rb, ps),
                    pltpu.make_async_copy(col_hbm.at[q], cb, ps),
                    pltpu.make_async_copy(ew_hbm.at[q], eb, ps)]

        def g_desc(j, rb, gbuf, sem):
            return pltpu.make_async_copy(src.at[rb.at[j]], gbuf, sem)

        def s_desc(j, cb, gbuf, sem):
            return pltpu.make_async_copy(gbuf, acc.at[cb.at[j]], sem)

        def scale(j, eb, gbuf):
            def edge4(e4, ecarry):
                for u in range(4):
                    e = e4 * 4 + u
                    ev = jnp.full((16,), j * EPB + e, jnp.int32)
                    w = plsc.load_gather(eb, [ev])
                    for f in range(8):
                        gbuf[e, pl.ds(f * 16, 16)] = gbuf[e, pl.ds(f * 16, 16)] * w
                return ecarry

            lax.fori_loop(0, EPB // 4, edge4, 0)

        def process_mega(bufs, nbufs, qn, cross):
            rb, cb, eb = bufs
            nrb, _, _ = nbufs

            def pair(p, carry):
                j0 = 2 * p
                j1 = j0 + 1
                g_desc(j0, rb, gb0, gs0).wait()

                @pl.when(p > 0)
                def _():
                    s_desc(j1 - 2, cb, gb1, ss1).wait()
                g_desc(j1, rb, gb1, gs1).start()
                scale(j0, eb, gb0)
                s_desc(j0, cb, gb0, ss0).start(add=True)
                g_desc(j1, rb, gb1, gs1).wait()
                s_desc(j0, cb, gb0, ss0).wait()

                @pl.when(p < pairs - 1)
                def _():
                    g_desc(j0 + 2, rb, gb0, gs0).start()

                @pl.when((p == pairs - 1) & cross)
                def _():
                    for d in idx_descs(qn, *nbufs):
                        d.wait()
                    g_desc(0, nrb, gb0, gs0).start()
                scale(j1, eb, gb1)
                s_desc(j1, cb, gb1, ss1).start(add=True)
                return carry

            lax.fori_loop(0, pairs, pair, 0)
            s_desc(MCH - 1, cb, gb1, ss1).wait()

        A = (rowa, cola, ewa)
        B = (rowq, colq, ewq)
        for d in idx_descs(qbase, *A):
            d.start()

        def zrow(r, carry):
            for f in range(8):
                ib[r, pl.ds(f * 16, 16)] = jnp.zeros((16,), jnp.float32)
            return carry

        lax.fori_loop(0, RP_BLK, zrow, 0)

        def zinit(i, carry):
            pltpu.sync_copy(ib, acc.at[pl.ds(s * RP_S + i * RP_BLK, RP_BLK)])
            return carry

        lax.fori_loop(0, RP_S // RP_BLK, zinit, 0)
        for d in idx_descs(qbase, *A):
            d.wait()
        g_desc(0, rowa, gb0, gs0).start()
        for d in idx_descs(qbase + 1, *B):
            d.start()
        plsc.subcore_barrier()

        true_ = jnp.bool_(True)

        def megapair(i, carry):
            m0 = 2 * i
            process_mega(A, B, qbase + m0 + 1, true_)

            @pl.when(m0 + 2 < nmega)
            def _():
                for d in idx_descs(qbase + m0 + 2, *A):
                    d.start()
            process_mega(B, A, qbase + m0 + 2, m0 + 2 < nmega)

            @pl.when(m0 + 3 < nmega)
            def _():
                for d in idx_descs(qbase + m0 + 3, *B):
                    d.start()
            return carry

        lax.fori_loop(0, nmega // 2, megapair, 0)
        plsc.subcore_barrier()

        def drain(i, carry):
            r0 = s * RP_S + i * RP_BLK
            pltpu.sync_copy(acc.at[pl.ds(r0, RP_BLK)],
                            out_hbm.at[c].at[pl.ds(r0, RP_BLK)])
            return carry

        lax.fori_loop(0, RP_S // RP_BLK, drain, 0)

    return k


_scatter_l1 = _make_scatter(True, 20)
_scatter_l2 = _make_scatter(False, 10)

_BN = 1024


def _tc1_body(x_ref, w1_ref, deg_ref, xs_ref, dinv_ref):
    deg = deg_ref[...]
    dinv = lax.rsqrt((deg[0:1, :] + deg[1:2, :]).T + 1.0)
    dinv_ref[...] = dinv
    xw = jnp.dot(x_ref[...], w1_ref[...], preferred_element_type=jnp.float32)
    xs = xw * dinv
    xs_ref[0] = xs[:, :128]
    xs_ref[1] = xs[:, 128:]


def _tc1(x, W1, degp):
    return pl.pallas_call(
        _tc1_body,
        grid=(NP // _BN,),
        in_specs=[
            pl.BlockSpec((_BN, 128), lambda i: (i, 0)),
            pl.BlockSpec((128, 256), lambda i: (0, 0)),
            pl.BlockSpec((2, _BN), lambda i: (0, i)),
        ],
        out_specs=[
            pl.BlockSpec((2, _BN, 128), lambda i: (0, i, 0)),
            pl.BlockSpec((_BN, 1), lambda i: (i, 0)),
        ],
        out_shape=[
            jax.ShapeDtypeStruct((2, NP, 128), jnp.float32),
            jax.ShapeDtypeStruct((NP, 1), jnp.float32),
        ],
    )(x, W1, degp)


def _tc2_body(tmp_ref, xs_ref, dinv_ref, b1_ref, w2_ref, xs2_ref):
    dinv = dinv_ref[...]
    b1 = b1_ref[...]
    w2 = w2_ref[...]
    h_lo = jnp.maximum((tmp_ref[0] + xs_ref[0]) * dinv + b1[:128], 0.0)
    h_hi = jnp.maximum((tmp_ref[1] + xs_ref[1]) * dinv + b1[128:], 0.0)
    xw2 = (jnp.dot(h_lo, w2[:128], preferred_element_type=jnp.float32)
           + jnp.dot(h_hi, w2[128:], preferred_element_type=jnp.float32))
    xs2_ref[...] = xw2 * dinv


def _tc2(tmp, xs1, dinv, b1, W2):
    return pl.pallas_call(
        _tc2_body,
        grid=(NP // _BN,),
        in_specs=[
            pl.BlockSpec((2, _BN, 128), lambda i: (0, i, 0)),
            pl.BlockSpec((2, _BN, 128), lambda i: (0, i, 0)),
            pl.BlockSpec((_BN, 1), lambda i: (i, 0)),
            pl.BlockSpec((256,), lambda i: (0,)),
            pl.BlockSpec((256, 128), lambda i: (0, 0)),
        ],
        out_specs=pl.BlockSpec((_BN, 128), lambda i: (i, 0)),
        out_shape=jax.ShapeDtypeStruct((NP, 128), jnp.float32),
    )(tmp, xs1, dinv, b1, W2)


def _tc3_body(tmp_ref, xs_ref, dinv_ref, b2_ref, wl1_ref, bl1_ref, wl2_ref,
              bl2_ref, o_ref):
    dinv = dinv_ref[...]
    h = jnp.maximum((tmp_ref[0] + tmp_ref[1] + xs_ref[...]) * dinv
                    + b2_ref[...], 0.0)
    t = jnp.maximum(
        jnp.dot(h, wl1_ref[...], preferred_element_type=jnp.float32)
        + bl1_ref[...], 0.0)
    o_ref[...] = jnp.maximum(
        jnp.dot(t, wl2_ref[...], preferred_element_type=jnp.float32)
        + bl2_ref[...], 0.0)


def _tc3(tmp2, xs2, dinv, b2, Wl1, bl1, Wl2, bl2):
    return pl.pallas_call(
        _tc3_body,
        grid=(NP // _BN,),
        in_specs=[
            pl.BlockSpec((2, _BN, 128), lambda i: (0, i, 0)),
            pl.BlockSpec((_BN, 128), lambda i: (i, 0)),
            pl.BlockSpec((_BN, 1), lambda i: (i, 0)),
            pl.BlockSpec((128,), lambda i: (0,)),
            pl.BlockSpec((128, 512), lambda i: (0, 0)),
            pl.BlockSpec((512,), lambda i: (0,)),
            pl.BlockSpec((512, 128), lambda i: (0, 0)),
            pl.BlockSpec((128,), lambda i: (0,)),
        ],
        out_specs=pl.BlockSpec((_BN, 128), lambda i: (i, 0)),
        out_shape=jax.ShapeDtypeStruct((N, 128), jnp.float32),
    )(tmp2, xs2, dinv, b2, Wl1, bl1, Wl2, bl2)


def kernel(x, edge_index, edge_weight, W1, b1, W2, b2, Wl1, bl1, Wl2, bl2):
    row = edge_index[0]
    col = edge_index[1]
    colD = col.reshape(NC * NS, DEG_CH, DEG_EPB)
    ewD = edge_weight.reshape(NC * NS, DEG_CH * DEG_EPB)
    rowS = row.reshape(E // (MCH * EPB), MCH, EPB)
    colS = col.reshape(E // (MCH * EPB), MCH, EPB)
    ewS = edge_weight.reshape(E // (MCH * EPB), MCH * EPB)

    degp2 = _deg(colD, ewD)
    degp = degp2.reshape(NC, NP)

    x_p = jnp.pad(x, ((0, NP - N), (0, 0)))
    xs1, dinv = _tc1(x_p, W1, degp)
    tmp1 = _scatter_l1(xs1, rowS, colS, ewS)
    xs2 = _tc2(tmp1, xs1, dinv, b1, W2)
    tmp2 = _scatter_l2(xs2, rowS, colS, ewS)
    return _tc3(tmp2, xs2, dinv, b2, Wl1, bl1, Wl2, bl2)


# speedup vs baseline: 19.9010x; 1.0038x over previous
import functools

import jax
import jax.numpy as jnp
from jax import lax
from jax.experimental import pallas as pl
from jax.experimental.pallas import tpu as pltpu
from jax.experimental.pallas import tpu_sc as plsc

N = 10000
E = 320000
NC = 2
NS = 16
NP = 10240

DEG_CH, DEG_EPB = 125, 80
EPB = 100
MCH = 10
RP_S = NP // NS
RP_BLK = 64

_mesh = plsc.VectorSubcoreMesh(core_axis_name="c", subcore_axis_name="s")


@functools.partial(
    pl.kernel,
    out_type=jax.ShapeDtypeStruct((NC * NP,), jnp.float32),
    mesh=_mesh,
    compiler_params=pltpu.CompilerParams(needs_layout_passes=False),
    scratch_types=[
        pltpu.VMEM((DEG_CH, DEG_EPB), jnp.int32),
        pltpu.VMEM((DEG_CH * DEG_EPB,), jnp.float32),
        pltpu.VMEM((NP // NS,), jnp.float32),
        pltpu.VMEM_SHARED((NP,), jnp.float32),
    ],
)
def _deg(col_hbm, ew_hbm, deg_hbm, colb, ewb, zb, deg_sh):
    c = lax.axis_index("c")
    s = lax.axis_index("s")
    npw = NP // NS

    def zero(i, carry):
        zb[pl.ds(i * 16, 16)] = jnp.zeros((16,), jnp.float32)
        return carry

    lax.fori_loop(0, npw // 16, zero, 0)
    pltpu.sync_copy(zb, deg_sh.at[pl.ds(s * npw, npw)])
    plsc.subcore_barrier()

    q = c * NS + s
    pltpu.sync_copy(col_hbm.at[q], colb)
    pltpu.sync_copy(ew_hbm.at[q], ewb)

    def chunk(j, carry):
        pltpu.sync_copy(ewb.at[pl.ds(j * DEG_EPB, DEG_EPB)],
                        deg_sh.at[colb.at[j]], add=True)
        return carry

    lax.fori_loop(0, DEG_CH, chunk, 0)
    plsc.subcore_barrier()
    pltpu.sync_copy(deg_sh.at[pl.ds(s * npw, npw)], zb)
    pltpu.sync_copy(zb, deg_hbm.at[pl.ds(c * NP + s * npw, npw)])


def _make_scatter(split_features, nmega):
    epm = MCH * EPB
    pairs = MCH // 2

    @functools.partial(
        pl.kernel,
        out_type=jax.ShapeDtypeStruct((NC, NP, 128), jnp.float32),
        mesh=_mesh,
        compiler_params=pltpu.CompilerParams(needs_layout_passes=False),
        scratch_types=[
            pltpu.VMEM((MCH, EPB), jnp.int32),
            pltpu.VMEM((MCH, EPB), jnp.int32),
            pltpu.VMEM((epm,), jnp.float32),
            pltpu.VMEM((MCH, EPB), jnp.int32),
            pltpu.VMEM((MCH, EPB), jnp.int32),
            pltpu.VMEM((epm,), jnp.float32),
            pltpu.VMEM((EPB, 128), jnp.float32),
            pltpu.VMEM((EPB, 128), jnp.float32),
            pltpu.VMEM((RP_BLK, 128), jnp.float32),
            pltpu.VMEM_SHARED((NP, 128), jnp.float32),
            pltpu.SemaphoreType.DMA,
            pltpu.SemaphoreType.DMA,
            pltpu.SemaphoreType.DMA,
            pltpu.SemaphoreType.DMA,
            pltpu.SemaphoreType.DMA,
        ],
    )
    def k(xs_hbm, row_hbm, col_hbm, ew_hbm, out_hbm,
          rowa, cola, ewa, rowq, colq, ewq, gb0, gb1, ib, acc,
          gs0, gs1, ss0, ss1, ps):
        c = lax.axis_index("c")
        s = lax.axis_index("s")
        src = xs_hbm.at[c] if split_features else xs_hbm
        qbase = (s if split_features else c * NS + s) * nmega

        def idx_descs(q, rb, cb, eb):
            return [pltpu.make_async_copy(row_hbm.at[q], rb, ps),
                    pltpu.make_async_copy(col_hbm.at[q], cb, ps),
                    pltpu.make_async_copy(ew_hbm.at[q], eb, ps)]

        def g_desc(j, rb, gbuf, sem):
            return pltpu.make_async_copy(src.at[rb.at[j]], gbuf, sem)

        def s_desc(j, cb, gbuf, sem):
            return pltpu.make_async_copy(gbuf, acc.at[cb.at[j]], sem)

        def scale(j, eb, gbuf):
            def edge4(e4, ecarry):
                for u in range(4):
                    e = e4 * 4 + u
                    ev = jnp.full((16,), j * EPB + e, jnp.int32)
                    w = plsc.load_gather(eb, [ev])
                    for f in range(8):
                        gbuf[e, pl.ds(f * 16, 16)] = gbuf[e, pl.ds(f * 16, 16)] * w
                return ecarry

            lax.fori_loop(0, EPB // 4, edge4, 0)

        def process_mega(bufs, nbufs, qn, cross):
            rb, cb, eb = bufs
            nrb, _, _ = nbufs

            def pair(p, carry):
                j0 = 2 * p
                j1 = j0 + 1
                g_desc(j0, rb, gb0, gs0).wait()

                @pl.when(p > 0)
                def _():
                    s_desc(j1 - 2, cb, gb1, ss1).wait()
                g_desc(j1, rb, gb1, gs1).start()
                scale(j0, eb, gb0)
                s_desc(j0, cb, gb0, ss0).start(add=True)
                g_desc(j1, rb, gb1, gs1).wait()
                s_desc(j0, cb, gb0, ss0).wait()

                @pl.when(p < pairs - 1)
                def _():
                    g_desc(j0 + 2, rb, gb0, gs0).start()

                @pl.when((p == pairs - 1) & cross)
                def _():
                    for d in idx_descs(qn, *nbufs):
                        d.wait()
                    g_desc(0, nrb, gb0, gs0).start()
                scale(j1, eb, gb1)
                s_desc(j1, cb, gb1, ss1).start(add=True)
                return carry

            lax.fori_loop(0, pairs, pair, 0)
            s_desc(MCH - 1, cb, gb1, ss1).wait()

        A = (rowa, cola, ewa)
        B = (rowq, colq, ewq)
        for d in idx_descs(qbase, *A):
            d.start()

        def zrow(r, carry):
            for f in range(8):
                ib[r, pl.ds(f * 16, 16)] = jnp.zeros((16,), jnp.float32)
            return carry

        lax.fori_loop(0, RP_BLK, zrow, 0)

        def zdesc(i):
            return pltpu.make_async_copy(
                ib, acc.at[pl.ds(s * RP_S + i * RP_BLK, RP_BLK)], gs1)

        for i in range(RP_S // RP_BLK):
            zdesc(i).start()
        for i in range(RP_S // RP_BLK):
            zdesc(i).wait()
        for d in idx_descs(qbase, *A):
            d.wait()
        g_desc(0, rowa, gb0, gs0).start()
        for d in idx_descs(qbase + 1, *B):
            d.start()
        plsc.subcore_barrier()

        true_ = jnp.bool_(True)

        def megapair(i, carry):
            m0 = 2 * i
            process_mega(A, B, qbase + m0 + 1, true_)

            @pl.when(m0 + 2 < nmega)
            def _():
                for d in idx_descs(qbase + m0 + 2, *A):
                    d.start()
            process_mega(B, A, qbase + m0 + 2, m0 + 2 < nmega)

            @pl.when(m0 + 3 < nmega)
            def _():
                for d in idx_descs(qbase + m0 + 3, *B):
                    d.start()
            return carry

        lax.fori_loop(0, nmega // 2, megapair, 0)
        plsc.subcore_barrier()

        def ddesc(i):
            r0 = s * RP_S + i * RP_BLK
            return pltpu.make_async_copy(
                acc.at[pl.ds(r0, RP_BLK)],
                out_hbm.at[c].at[pl.ds(r0, RP_BLK)], gs1)

        for i in range(RP_S // RP_BLK):
            ddesc(i).start()
        for i in range(RP_S // RP_BLK):
            ddesc(i).wait()

    return k


_scatter_l1 = _make_scatter(True, 20)
_scatter_l2 = _make_scatter(False, 10)

_BN = 1024


def _tc1_body(x_ref, w1_ref, deg_ref, xs_ref, dinv_ref):
    deg = deg_ref[...]
    dinv = lax.rsqrt((deg[0:1, :] + deg[1:2, :]).T + 1.0)
    dinv_ref[...] = dinv
    xw = jnp.dot(x_ref[...], w1_ref[...], preferred_element_type=jnp.float32)
    xs = xw * dinv
    xs_ref[0] = xs[:, :128]
    xs_ref[1] = xs[:, 128:]


def _tc1(x, W1, degp):
    return pl.pallas_call(
        _tc1_body,
        grid=(NP // _BN,),
        in_specs=[
            pl.BlockSpec((_BN, 128), lambda i: (i, 0)),
            pl.BlockSpec((128, 256), lambda i: (0, 0)),
            pl.BlockSpec((2, _BN), lambda i: (0, i)),
        ],
        out_specs=[
            pl.BlockSpec((2, _BN, 128), lambda i: (0, i, 0)),
            pl.BlockSpec((_BN, 1), lambda i: (i, 0)),
        ],
        out_shape=[
            jax.ShapeDtypeStruct((2, NP, 128), jnp.float32),
            jax.ShapeDtypeStruct((NP, 1), jnp.float32),
        ],
    )(x, W1, degp)


def _tc2_body(tmp_ref, xs_ref, dinv_ref, b1_ref, w2_ref, xs2_ref):
    dinv = dinv_ref[...]
    b1 = b1_ref[...]
    w2 = w2_ref[...]
    h_lo = jnp.maximum((tmp_ref[0] + xs_ref[0]) * dinv + b1[:128], 0.0)
    h_hi = jnp.maximum((tmp_ref[1] + xs_ref[1]) * dinv + b1[128:], 0.0)
    xw2 = (jnp.dot(h_lo, w2[:128], preferred_element_type=jnp.float32)
           + jnp.dot(h_hi, w2[128:], preferred_element_type=jnp.float32))
    xs2_ref[...] = xw2 * dinv


def _tc2(tmp, xs1, dinv, b1, W2):
    return pl.pallas_call(
        _tc2_body,
        grid=(NP // _BN,),
        in_specs=[
            pl.BlockSpec((2, _BN, 128), lambda i: (0, i, 0)),
            pl.BlockSpec((2, _BN, 128), lambda i: (0, i, 0)),
            pl.BlockSpec((_BN, 1), lambda i: (i, 0)),
            pl.BlockSpec((256,), lambda i: (0,)),
            pl.BlockSpec((256, 128), lambda i: (0, 0)),
        ],
        out_specs=pl.BlockSpec((_BN, 128), lambda i: (i, 0)),
        out_shape=jax.ShapeDtypeStruct((NP, 128), jnp.float32),
    )(tmp, xs1, dinv, b1, W2)


def _tc3_body(tmp_ref, xs_ref, dinv_ref, b2_ref, wl1_ref, bl1_ref, wl2_ref,
              bl2_ref, o_ref):
    dinv = dinv_ref[...]
    h = jnp.maximum((tmp_ref[0] + tmp_ref[1] + xs_ref[...]) * dinv
                    + b2_ref[...], 0.0)
    t = jnp.maximum(
        jnp.dot(h, wl1_ref[...], preferred_element_type=jnp.float32)
        + bl1_ref[...], 0.0)
    o_ref[...] = jnp.maximum(
        jnp.dot(t, wl2_ref[...], preferred_element_type=jnp.float32)
        + bl2_ref[...], 0.0)


def _tc3(tmp2, xs2, dinv, b2, Wl1, bl1, Wl2, bl2):
    return pl.pallas_call(
        _tc3_body,
        grid=(NP // _BN,),
        in_specs=[
            pl.BlockSpec((2, _BN, 128), lambda i: (0, i, 0)),
            pl.BlockSpec((_BN, 128), lambda i: (i, 0)),
            pl.BlockSpec((_BN, 1), lambda i: (i, 0)),
            pl.BlockSpec((128,), lambda i: (0,)),
            pl.BlockSpec((128, 512), lambda i: (0, 0)),
            pl.BlockSpec((512,), lambda i: (0,)),
            pl.BlockSpec((512, 128), lambda i: (0, 0)),
            pl.BlockSpec((128,), lambda i: (0,)),
        ],
        out_specs=pl.BlockSpec((_BN, 128), lambda i: (i, 0)),
        out_shape=jax.ShapeDtypeStruct((N, 128), jnp.float32),
    )(tmp2, xs2, dinv, b2, Wl1, bl1, Wl2, bl2)


def kernel(x, edge_index, edge_weight, W1, b1, W2, b2, Wl1, bl1, Wl2, bl2):
    row = edge_index[0]
    col = edge_index[1]
    colD = col.reshape(NC * NS, DEG_CH, DEG_EPB)
    ewD = edge_weight.reshape(NC * NS, DEG_CH * DEG_EPB)
    rowS = row.reshape(E // (MCH * EPB), MCH, EPB)
    colS = col.reshape(E // (MCH * EPB), MCH, EPB)
    ewS = edge_weight.reshape(E // (MCH * EPB), MCH * EPB)

    degp2 = _deg(colD, ewD)
    degp = degp2.reshape(NC, NP)

    x_p = jnp.pad(x, ((0, NP - N), (0, 0)))
    xs1, dinv = _tc1(x_p, W1, degp)
    tmp1 = _scatter_l1(xs1, rowS, colS, ewS)
    xs2 = _tc2(tmp1, xs1, dinv, b1, W2)
    tmp2 = _scatter_l2(xs2, rowS, colS, ewS)
    return _tc3(tmp2, xs2, dinv, b2, Wl1, bl1, Wl2, bl2)
